# baseline (speedup 1.0000x reference)
import jax
import jax.numpy as jnp
from jax import lax
from jax.experimental import pallas as pl
from jax.experimental.pallas import tpu as pltpu
from jax.experimental.pallas import tpu_sc as plsc

_B = 16384
_N_ORD = 9
_NW = 32
_CHUNK = _B // _NW
_NG = _CHUNK // 16
_XDIM = 56
_TTAB = 8008
_OTAB = 396


def _sc_gather_body(text_h, ordf_h, ohf_h, numf_h, ttab_h, otab_h, out_h,
                    text_v, ord_v, oh_v, num_v, ttab_v, otab_v, out_v):
    nc = jax.lax.axis_size("c")
    wid = lax.axis_index("s") * nc + lax.axis_index("c")
    base = wid * _CHUNK

    pltpu.sync_copy(text_h.at[pl.ds(base, _CHUNK)], text_v)
    pltpu.sync_copy(ordf_h.at[pl.ds(base * _N_ORD, _CHUNK * _N_ORD)], ord_v)
    pltpu.sync_copy(ohf_h.at[pl.ds(base * 10, _CHUNK * 10)], oh_v)
    pltpu.sync_copy(numf_h.at[pl.ds(base * 2, _CHUNK * 2)], num_v)
    pltpu.sync_copy(ttab_h, ttab_v)
    pltpu.sync_copy(otab_h, otab_v)

    lane = lax.iota(jnp.int32, 16)

    def group(g, carry):
        r0 = g * 16
        rvec = lane + r0
        tvec = text_v[pl.ds(r0, 16)]
        tfi = tvec * 8
        for j in range(8):
            out_v[j, pl.ds(r0, 16)] = plsc.load_gather(ttab_v, [tfi + j])
        r9 = rvec * _N_ORD
        for i in range(_N_ORD):
            ovec = plsc.load_gather(ord_v, [r9 + i])
            ofi = ovec * 4 + (i * 44)
            for j in range(4):
                out_v[8 + 4 * i + j, pl.ds(r0, 16)] = plsc.load_gather(
                    otab_v, [ofi + j])
        r10 = rvec * 10
        for c in range(10):
            out_v[44 + c, pl.ds(r0, 16)] = plsc.load_gather(oh_v, [r10 + c])
        r2 = rvec * 2
        for c in range(2):
            out_v[54 + c, pl.ds(r0, 16)] = plsc.load_gather(num_v, [r2 + c])
        return carry

    lax.fori_loop(0, _NG, group, 0)
    pltpu.sync_copy(out_v, out_h.at[wid])


def _sc_gather(text, ordf, ohf, numf, ttabf, otabf):
    mesh = plsc.VectorSubcoreMesh(core_axis_name="c", subcore_axis_name="s")
    fn = pl.kernel(
        _sc_gather_body,
        out_type=jax.ShapeDtypeStruct((_NW, _XDIM, _CHUNK), jnp.float32),
        mesh=mesh,
        compiler_params=pltpu.CompilerParams(needs_layout_passes=False),
        scratch_types=[
            pltpu.VMEM((_CHUNK,), jnp.int32),
            pltpu.VMEM((_CHUNK * _N_ORD,), jnp.int32),
            pltpu.VMEM((_CHUNK * 10,), jnp.float32),
            pltpu.VMEM((_CHUNK * 2,), jnp.float32),
            pltpu.VMEM((_TTAB,), jnp.float32),
            pltpu.VMEM((_OTAB,), jnp.float32),
            pltpu.VMEM((_XDIM, _CHUNK), jnp.float32),
        ],
    )
    return fn(text, ordf, ohf, numf, ttabf, otabf)


def _mlp_body(x_ref, w1_ref, b1_ref, w2_ref, b2_ref, w3_ref, b3_ref, o_ref):
    xT = x_ref[0]
    h1 = lax.dot_general(w1_ref[...], xT, (((0,), (0,)), ((), ())),
                         preferred_element_type=jnp.float32)
    h1 = jnp.maximum(h1 + b1_ref[...], 0.0)
    h2 = lax.dot_general(w2_ref[...], h1, (((0,), (0,)), ((), ())),
                         preferred_element_type=jnp.float32)
    h2 = jnp.maximum(h2 + b2_ref[...], 0.0)
    o = lax.dot_general(h2, w3_ref[...], (((0,), (0,)), ((), ())),
                        preferred_element_type=jnp.float32)
    o_ref[0] = o + b3_ref[...]


def _mlp(x3, W1, b1c, W2, b2c, W3, b3c):
    return pl.pallas_call(
        _mlp_body,
        grid=(_NW,),
        in_specs=[
            pl.BlockSpec((1, _XDIM, _CHUNK), lambda i: (i, 0, 0)),
            pl.BlockSpec((_XDIM, 128), lambda i: (0, 0)),
            pl.BlockSpec((128, 1), lambda i: (0, 0)),
            pl.BlockSpec((128, 64), lambda i: (0, 0)),
            pl.BlockSpec((64, 1), lambda i: (0, 0)),
            pl.BlockSpec((64, 3), lambda i: (0, 0)),
            pl.BlockSpec((1, 3), lambda i: (0, 0)),
        ],
        out_specs=pl.BlockSpec((1, _CHUNK, 3), lambda i: (i, 0, 0)),
        out_shape=jax.ShapeDtypeStruct((_NW, _CHUNK, 3), jnp.float32),
    )(x3, W1, b1c, W2, b2c, W3, b3c)


def kernel(text, ord, onehot, num, text_table, ord_tables, W1, b1, W2, b2, W3, b3):
    text = text.astype(jnp.int32)
    ordf = ord.astype(jnp.int32).reshape(-1)
    ohf = onehot.reshape(-1)
    numf = num.reshape(-1)
    ttabf = text_table.reshape(-1)
    otabf = ord_tables.reshape(-1)

    x3 = _sc_gather(text, ordf, ohf, numf, ttabf, otabf)
    return x3[:, 0:3, :].transpose(0, 2, 1).reshape(_B, 3)


# speedup vs baseline: 1.1216x; 1.1216x over previous
import jax
import jax.numpy as jnp
from jax import lax
from jax.experimental import pallas as pl
from jax.experimental.pallas import tpu as pltpu
from jax.experimental.pallas import tpu_sc as plsc

_B = 16384
_N_ORD = 9
_NW = 32
_CHUNK = _B // _NW
_NG = _CHUNK // 16
_XROWS = 48
_TTAB = 8008
_OTAB = 396
_BM = 2048
_GRID = _B // _BM


def _sc_gather_body(text_h, ord_h, ttab_h, otab_h, out_h,
                    text_v, ord_v, ttab_v, otab_v, out_v,
                    sem0, sem1, sem2, sem3):
    nc = jax.lax.axis_size("c")
    wid = lax.axis_index("s") * nc + lax.axis_index("c")
    base = wid * _CHUNK

    c0 = pltpu.async_copy(text_h.at[pl.ds(base, _CHUNK)], text_v, sem0)
    c1 = pltpu.async_copy(ord_h.at[pl.ds(base, _CHUNK)], ord_v, sem1)
    c2 = pltpu.async_copy(ttab_h, ttab_v, sem2)
    c3 = pltpu.async_copy(otab_h, otab_v, sem3)
    c0.wait()
    c1.wait()
    c2.wait()
    c3.wait()

    lane = lax.iota(jnp.int32, 16)

    def group(g, carry):
        r0 = g * 16
        rvec = lane + r0
        tvec = text_v[pl.ds(r0, 16)]
        tfi = tvec * 8
        for j in range(8):
            out_v[j, pl.ds(r0, 16)] = plsc.load_gather(ttab_v, [tfi + j])
        for i in range(_N_ORD):
            ovec = plsc.load_gather(ord_v, [rvec, jnp.full((16,), i, jnp.int32)])
            ofi = ovec * 4 + (i * 44)
            for j in range(4):
                out_v[8 + 4 * i + j, pl.ds(r0, 16)] = plsc.load_gather(
                    otab_v, [ofi + j])
        return carry

    lax.fori_loop(0, _NG, group, 0)
    pltpu.sync_copy(out_v, out_h.at[:, pl.ds(base, _CHUNK)])


def _sc_gather(text, ord2d, ttabf, otabf):
    mesh = plsc.VectorSubcoreMesh(core_axis_name="c", subcore_axis_name="s")
    fn = pl.kernel(
        _sc_gather_body,
        out_type=jax.ShapeDtypeStruct((_XROWS, _B), jnp.float32),
        mesh=mesh,
        compiler_params=pltpu.CompilerParams(needs_layout_passes=False),
        scratch_types=[
            pltpu.VMEM((_CHUNK,), jnp.int32),
            pltpu.VMEM((_CHUNK, _N_ORD), jnp.int32),
            pltpu.VMEM((_TTAB,), jnp.float32),
            pltpu.VMEM((_OTAB,), jnp.float32),
            pltpu.VMEM((_XROWS, _CHUNK), jnp.float32),
            pltpu.SemaphoreType.DMA,
            pltpu.SemaphoreType.DMA,
            pltpu.SemaphoreType.DMA,
            pltpu.SemaphoreType.DMA,
        ],
    )
    return fn(text, ord2d, ttabf, otabf)


def _mlp_body(x_ref, oh_ref, num_ref, w1_ref, b1_ref, w2_ref, b2_ref,
              w3_ref, b3_ref, o_ref):
    w1 = w1_ref[...]
    xT = x_ref[pl.ds(0, 44), :]
    h1 = lax.dot_general(w1[0:44, :], xT, (((0,), (0,)), ((), ())),
                         preferred_element_type=jnp.float32)
    h1 = h1 + lax.dot_general(w1[44:54, :], oh_ref[...],
                              (((0,), (1,)), ((), ())),
                              preferred_element_type=jnp.float32)
    h1 = h1 + lax.dot_general(w1[54:56, :], num_ref[...],
                              (((0,), (1,)), ((), ())),
                              preferred_element_type=jnp.float32)
    h1 = jnp.maximum(h1 + b1_ref[...], 0.0)
    h2 = lax.dot_general(w2_ref[...], h1, (((0,), (0,)), ((), ())),
                         preferred_element_type=jnp.float32)
    h2 = jnp.maximum(h2 + b2_ref[...], 0.0)
    o = lax.dot_general(h2, w3_ref[...], (((0,), (0,)), ((), ())),
                        preferred_element_type=jnp.float32)
    o_ref[...] = o + b3_ref[...]


def _mlp(x2, onehot, num, W1, b1c, W2, b2c, W3, b3c):
    return pl.pallas_call(
        _mlp_body,
        grid=(_GRID,),
        in_specs=[
            pl.BlockSpec((_XROWS, _BM), lambda i: (0, i)),
            pl.BlockSpec((_BM, 10), lambda i: (i, 0)),
            pl.BlockSpec((_BM, 2), lambda i: (i, 0)),
            pl.BlockSpec((56, 128), lambda i: (0, 0)),
            pl.BlockSpec((128, 1), lambda i: (0, 0)),
            pl.BlockSpec((128, 64), lambda i: (0, 0)),
            pl.BlockSpec((64, 1), lambda i: (0, 0)),
            pl.BlockSpec((64, 3), lambda i: (0, 0)),
            pl.BlockSpec((1, 3), lambda i: (0, 0)),
        ],
        out_specs=pl.BlockSpec((_BM, 3), lambda i: (i, 0)),
        out_shape=jax.ShapeDtypeStruct((_B, 3), jnp.float32),
    )(x2, onehot, num, W1, b1c, W2, b2c, W3, b3c)


def kernel(text, ord, onehot, num, text_table, ord_tables, W1, b1, W2, b2, W3, b3):
    text = text.astype(jnp.int32)
    ord2d = ord.astype(jnp.int32)
    ttabf = text_table.reshape(-1)
    otabf = ord_tables.reshape(-1)

    x2 = _sc_gather(text, ord2d, ttabf, otabf)

    return _mlp(x2, onehot, num, W1, b1.reshape(128, 1), W2,
                b2.reshape(64, 1), W3, b3.reshape(1, 3))


# speedup vs baseline: 1.2975x; 1.1568x over previous
import jax
import jax.numpy as jnp
from jax import lax
from jax.experimental import pallas as pl
from jax.experimental.pallas import tpu as pltpu
from jax.experimental.pallas import tpu_sc as plsc

_B = 16384
_N_ORD = 9
_NW = 32
_CHUNK = _B // _NW
_NG = _CHUNK // 16
_XROWS = 48
_TTAB = 8008
_OTAB = 396
_BM = 4096
_GRID = _B // _BM


def _sc_gather_body(text_h, ord_h, ttab_h, otab_h, out_h,
                    text_v, ord_v, ttab_v, otab_v, out_v,
                    sem0, sem1, sem2, sem3):
    nc = jax.lax.axis_size("c")
    wid = lax.axis_index("s") * nc + lax.axis_index("c")
    base = wid * _CHUNK

    c0 = pltpu.async_copy(text_h.at[pl.ds(base, _CHUNK)], text_v, sem0)
    c1 = pltpu.async_copy(ord_h.at[pl.ds(base, _CHUNK)], ord_v, sem1)
    c2 = pltpu.async_copy(ttab_h, ttab_v, sem2)
    c3 = pltpu.async_copy(otab_h, otab_v, sem3)
    c0.wait()
    c1.wait()
    c2.wait()
    c3.wait()

    lane = lax.iota(jnp.int32, 16)

    @plsc.parallel_loop(0, _NG, 1, unroll=2)
    def group(g):
        r0 = g * 16
        rvec = lane + r0
        tvec = text_v[pl.ds(r0, 16)]
        tfi = tvec * 8
        for j in range(8):
            out_v[j, pl.ds(r0, 16)] = plsc.load_gather(ttab_v, [tfi + j])
        for i in range(_N_ORD):
            ovec = plsc.load_gather(ord_v, [rvec, jnp.full((16,), i, jnp.int32)])
            ofi = ovec * 4 + (i * 44)
            for j in range(4):
                out_v[8 + 4 * i + j, pl.ds(r0, 16)] = plsc.load_gather(
                    otab_v, [ofi + j])
    pltpu.sync_copy(out_v, out_h.at[:, pl.ds(base, _CHUNK)])


def _sc_gather(text, ord2d, ttabf, otabf):
    mesh = plsc.VectorSubcoreMesh(core_axis_name="c", subcore_axis_name="s")
    fn = pl.kernel(
        _sc_gather_body,
        out_type=jax.ShapeDtypeStruct((_XROWS, _B), jnp.float32),
        mesh=mesh,
        compiler_params=pltpu.CompilerParams(needs_layout_passes=False),
        scratch_types=[
            pltpu.VMEM((_CHUNK,), jnp.int32),
            pltpu.VMEM((_CHUNK, _N_ORD), jnp.int32),
            pltpu.VMEM((_TTAB,), jnp.float32),
            pltpu.VMEM((_OTAB,), jnp.float32),
            pltpu.VMEM((_XROWS, _CHUNK), jnp.float32),
            pltpu.SemaphoreType.DMA,
            pltpu.SemaphoreType.DMA,
            pltpu.SemaphoreType.DMA,
            pltpu.SemaphoreType.DMA,
        ],
    )
    return fn(text, ord2d, ttabf, otabf)


def _mlp_body(x_ref, oh_ref, num_ref, w1_ref, b1_ref, w2_ref, b2_ref,
              w3_ref, b3_ref, o_ref):
    w1 = w1_ref[...]
    xT = x_ref[pl.ds(0, 44), :]
    h1 = lax.dot_general(w1[0:44, :], xT, (((0,), (0,)), ((), ())),
                         preferred_element_type=jnp.float32)
    h1 = h1 + lax.dot_general(w1[44:54, :], oh_ref[...],
                              (((0,), (1,)), ((), ())),
                              preferred_element_type=jnp.float32)
    h1 = h1 + lax.dot_general(w1[54:56, :], num_ref[...],
                              (((0,), (1,)), ((), ())),
                              preferred_element_type=jnp.float32)
    h1 = jnp.maximum(h1 + b1_ref[...], 0.0)
    h2 = lax.dot_general(w2_ref[...], h1, (((0,), (0,)), ((), ())),
                         preferred_element_type=jnp.float32)
    h2 = jnp.maximum(h2 + b2_ref[...], 0.0)
    o = lax.dot_general(h2, w3_ref[...], (((0,), (0,)), ((), ())),
                        preferred_element_type=jnp.float32)
    o_ref[...] = o + b3_ref[...]


def _mlp(x2, onehot, num, W1, b1c, W2, b2c, W3, b3c):
    return pl.pallas_call(
        _mlp_body,
        grid=(_GRID,),
        in_specs=[
            pl.BlockSpec((_XROWS, _BM), lambda i: (0, i)),
            pl.BlockSpec((_BM, 10), lambda i: (i, 0)),
            pl.BlockSpec((_BM, 2), lambda i: (i, 0)),
            pl.BlockSpec((56, 128), lambda i: (0, 0)),
            pl.BlockSpec((128, 1), lambda i: (0, 0)),
            pl.BlockSpec((128, 64), lambda i: (0, 0)),
            pl.BlockSpec((64, 1), lambda i: (0, 0)),
            pl.BlockSpec((64, 3), lambda i: (0, 0)),
            pl.BlockSpec((1, 3), lambda i: (0, 0)),
        ],
        out_specs=pl.BlockSpec((_BM, 3), lambda i: (i, 0)),
        out_shape=jax.ShapeDtypeStruct((_B, 3), jnp.float32),
    )(x2, onehot, num, W1, b1c, W2, b2c, W3, b3c)


def kernel(text, ord, onehot, num, text_table, ord_tables, W1, b1, W2, b2, W3, b3):
    text = text.astype(jnp.int32)
    ord2d = ord.astype(jnp.int32)
    ttabf = text_table.reshape(-1)
    otabf = ord_tables.reshape(-1)

    x2 = _sc_gather(text, ord2d, ttabf, otabf)

    return _mlp(x2, onehot, num, W1, b1.reshape(128, 1), W2,
                b2.reshape(64, 1), W3, b3.reshape(1, 3))
